# baseline (speedup 1.0000x reference)
import jax
import jax.numpy as jnp
from jax.experimental import pallas as pl


def _copy_kernel(x_ref, o_ref):
    o_ref[...] = x_ref[...]


def kernel(x, edge_index, edge_attr, bn_gamma, bn_beta, eW, W1, b1, W2, b2):
    src = edge_index[0]
    dst = edge_index[1]
    n_nodes = x.shape[0]
    h = x
    L = bn_gamma.shape[0]
    for l in range(L):
        mu = jnp.mean(h, axis=0)
        var = jnp.var(h, axis=0)
        hb = (h - mu) / jnp.sqrt(var + 1e-5) * bn_gamma[l] + bn_beta[l]
        edge_emb = edge_attr @ eW[l]
        msg = jax.nn.relu(hb[src] + edge_emb)
        aggr = jax.ops.segment_sum(msg, dst, num_segments=n_nodes)
        pre = hb + aggr
        hh = jax.nn.relu(pre @ W1[l] + b1[l]) @ W2[l] + b2[l]
        if l < L - 1:
            hh = jax.nn.relu(hh)
        h = hh
    return pl.pallas_call(
        _copy_kernel,
        out_shape=jax.ShapeDtypeStruct(h.shape, h.dtype),
    )(h)


# speedup vs baseline: 1.1767x; 1.1767x over previous
import functools

import jax
import jax.numpy as jnp
from jax import lax
from jax.experimental import pallas as pl
from jax.experimental.pallas import tpu as pltpu
from jax.experimental.pallas import tpu_sc as plsc

N = 10000
E = 160000
D = 256
HALF = 128
ED = 16

NC = 2
NS = 16
EB = 80
EDGES_PER_SUB = E // NS
CHUNKS = EDGES_PER_SUB // EB
N_PAD = 10240
ROWS_PER_SUB = N_PAD // NS

EE_BLK = 2000
MLP_BLK = 1000


def _bn_body(h_ref, g_ref, b_ref, hb_ref, hb2_ref):
    h = h_ref[...]
    mu = jnp.mean(h, axis=0, keepdims=True)
    var = jnp.mean((h - mu) * (h - mu), axis=0, keepdims=True)
    hb = (h - mu) * lax.rsqrt(var + 1e-5) * g_ref[...] + b_ref[...]
    hb_ref[...] = hb
    hb2_ref[0] = hb[:, :HALF]
    hb2_ref[1] = hb[:, HALF:]


def _bn(h, gamma, beta):
    return pl.pallas_call(
        _bn_body,
        out_shape=[
            jax.ShapeDtypeStruct((N, D), jnp.float32),
            jax.ShapeDtypeStruct((NC, N, HALF), jnp.float32),
        ],
    )(h, gamma, beta)


def _ee_body(ea_ref, w_ref, o_ref):
    o_ref[0] = jnp.dot(ea_ref[...], w_ref[0],
                       preferred_element_type=jnp.float32)


def _edge_emb(edge_attr, w_halves):
    return pl.pallas_call(
        _ee_body,
        grid=(NC, E // EE_BLK),
        in_specs=[
            pl.BlockSpec((EE_BLK, ED), lambda c, e: (e, 0)),
            pl.BlockSpec((1, ED, HALF), lambda c, e: (c, 0, 0)),
        ],
        out_specs=pl.BlockSpec((1, EE_BLK, HALF), lambda c, e: (c, e, 0)),
        out_shape=jax.ShapeDtypeStruct((NC, E, HALF), jnp.float32),
    )(edge_attr, w_halves)


_sc_mesh = plsc.VectorSubcoreMesh(
    core_axis_name="c", subcore_axis_name="s", num_cores=NC, num_subcores=NS)


@functools.partial(
    pl.kernel,
    out_type=jax.ShapeDtypeStruct((NC, N_PAD, HALF), jnp.float32),
    mesh=_sc_mesh,
    scratch_types=[
        pltpu.VMEM((EB,), jnp.int32),
        pltpu.VMEM((EB,), jnp.int32),
        pltpu.VMEM((EB,), jnp.int32),
        pltpu.VMEM((EB, HALF), jnp.float32),
        pltpu.VMEM((EB, HALF), jnp.float32),
        pltpu.VMEM((EB, HALF), jnp.float32),
        pltpu.VMEM_SHARED((N_PAD, HALF), jnp.float32),
        pltpu.SemaphoreType.DMA,
    ],
)
def _edge_aggr(hb2, srcl, dstl, ee, out,
               src_v, dst_v, zi_v, gat_v, ee_v, z_v, aggr_sh, sem):
    c = lax.axis_index("c")
    s = lax.axis_index("s")

    zero16 = jnp.zeros((16,), jnp.float32)
    iota16 = lax.iota(jnp.int32, 16)

    def zb(i, _):
        z_v[i // 8, pl.ds((i % 8) * 16, 16)] = zero16
        return 0

    lax.fori_loop(0, EB * 8, zb, 0)

    def zshot(g, _):
        base = s * ROWS_PER_SUB + g * EB
        for j in range(EB // 16):
            zi_v[pl.ds(16 * j, 16)] = iota16 + (base + 16 * j)
        pltpu.sync_copy(z_v, aggr_sh.at[zi_v])
        return 0

    lax.fori_loop(0, ROWS_PER_SUB // EB, zshot, 0)
    plsc.subcore_barrier()

    def chunk(g, _):
        base = s * EDGES_PER_SUB + g * EB
        pltpu.sync_copy(srcl.at[pl.ds(base, EB)], src_v)
        pltpu.sync_copy(dstl.at[pl.ds(base, EB)], dst_v)
        pltpu.async_copy(hb2.at[c].at[src_v], gat_v, sem).wait()
        pltpu.sync_copy(ee.at[c, pl.ds(base, EB)], ee_v)

        def cbody(i, _):
            r = i // 8
            col = (i % 8) * 16
            v = gat_v[r, pl.ds(col, 16)] + ee_v[r, pl.ds(col, 16)]
            gat_v[r, pl.ds(col, 16)] = jnp.maximum(v, 0.0)
            return 0

        lax.fori_loop(0, EB * 8, cbody, 0)
        pltpu.sync_copy(gat_v, aggr_sh.at[dst_v], add=True)
        return 0

    lax.fori_loop(0, CHUNKS, chunk, 0)
    plsc.subcore_barrier()
    pltpu.sync_copy(aggr_sh.at[pl.ds(s * ROWS_PER_SUB, ROWS_PER_SUB)],
                    out.at[c, pl.ds(s * ROWS_PER_SUB, ROWS_PER_SUB)])


def _mlp_body(hb_ref, a_ref, w1_ref, b1_ref, w2_ref, b2_ref, o_ref, *, last):
    pre0 = hb_ref[:, :HALF] + a_ref[0]
    pre1 = hb_ref[:, HALF:] + a_ref[1]
    pre = jnp.concatenate([pre0, pre1], axis=1)
    h1 = jnp.maximum(
        jnp.dot(pre, w1_ref[...], preferred_element_type=jnp.float32)
        + b1_ref[...], 0.0)
    out = jnp.dot(h1, w2_ref[...], preferred_element_type=jnp.float32) \
        + b2_ref[...]
    if not last:
        out = jnp.maximum(out, 0.0)
    o_ref[...] = out


def _mlp(hb, aggr, w1, b1, w2, b2, last):
    return pl.pallas_call(
        functools.partial(_mlp_body, last=last),
        grid=(N // MLP_BLK,),
        in_specs=[
            pl.BlockSpec((MLP_BLK, D), lambda i: (i, 0)),
            pl.BlockSpec((NC, MLP_BLK, HALF), lambda i: (0, i, 0)),
            pl.BlockSpec((D, 2 * D), lambda i: (0, 0)),
            pl.BlockSpec((1, 2 * D), lambda i: (0, 0)),
            pl.BlockSpec((2 * D, D), lambda i: (0, 0)),
            pl.BlockSpec((1, D), lambda i: (0, 0)),
        ],
        out_specs=pl.BlockSpec((MLP_BLK, D), lambda i: (i, 0)),
        out_shape=jax.ShapeDtypeStruct((N, D), jnp.float32),
    )(hb, aggr, w1, b1, w2, b2)


def kernel(x, edge_index, edge_attr, bn_gamma, bn_beta, eW, W1, b1, W2, b2):
    src = edge_index[0].astype(jnp.int32)
    dst = edge_index[1].astype(jnp.int32)
    h = x
    L = bn_gamma.shape[0]
    for l in range(L):
        hb, hb2 = _bn(h, bn_gamma[l].reshape(1, D), bn_beta[l].reshape(1, D))
        w_halves = jnp.stack([eW[l][:, :HALF], eW[l][:, HALF:]])
        ee = _edge_emb(edge_attr, w_halves)
        aggr = _edge_aggr(hb2, src, dst, ee)
        h = _mlp(hb, aggr, W1[l], b1[l].reshape(1, 2 * D),
                 W2[l], b2[l].reshape(1, D), last=(l == L - 1))
    return h
